# baseline (speedup 1.0000x reference)
import jax
import jax.numpy as jnp
from jax import lax
from jax.experimental import pallas as pl
from jax.experimental.pallas import tpu as pltpu

V = 100000
H = 128
E = 256
B = 1024

TVF = 6272
NF = pl.cdiv(V, TVF)


def _fused_body(ids_ref, e_ref, w_ref, b_ref, out_ref, hid_ref):
    @pl.when(pl.program_id(0) == 0)
    def _():
        ids = ids_ref[...]
        iota = lax.broadcasted_iota(jnp.int32, (B, E), 1)
        oh = (ids == iota).astype(jnp.bfloat16)
        hid_ref[...] = jnp.dot(
            oh, e_ref[...].astype(jnp.bfloat16),
            preferred_element_type=jnp.float32).astype(jnp.bfloat16)

    out_ref[...] = lax.dot_general(
        hid_ref[...], w_ref[...].astype(jnp.bfloat16),
        (((1,), (1,)), ((), ())),
        preferred_element_type=jnp.float32,
    ) + b_ref[...]


def kernel(input_ids, embed_table, W_lm, b_lm):
    out = pl.pallas_call(
        _fused_body,
        grid=(NF,),
        in_specs=[
            pl.BlockSpec((B, 1), lambda j: (0, 0)),
            pl.BlockSpec((E, H), lambda j: (0, 0)),
            pl.BlockSpec((TVF, H), lambda j: (j, 0)),
            pl.BlockSpec((1, TVF), lambda j: (0, j)),
        ],
        out_specs=pl.BlockSpec((B, TVF), lambda j: (0, j)),
        out_shape=jax.ShapeDtypeStruct((B, V), jnp.float32),
        scratch_shapes=[pltpu.VMEM((B, H), jnp.bfloat16)],
    )(input_ids.reshape(B, 1).astype(jnp.int32), embed_table, W_lm,
      b_lm.reshape(1, V))
    return out.reshape(32, 32, V)


# speedup vs baseline: 1.0123x; 1.0123x over previous
import jax
import jax.numpy as jnp
from jax import lax
from jax.experimental import pallas as pl
from jax.experimental.pallas import tpu as pltpu

V = 100000
H = 128
E = 256
B = 1024

TVF = 5632
NF = pl.cdiv(V, TVF)


def _fused_body(ids_ref, e_ref, w_ref, b_ref, out_ref, hid_ref):
    @pl.when(pl.program_id(0) == 0)
    def _():
        ids = ids_ref[...]
        iota = lax.broadcasted_iota(jnp.int32, (B, E), 1)
        oh = (ids == iota).astype(jnp.bfloat16)
        hid_ref[...] = jnp.dot(
            oh, e_ref[...].astype(jnp.bfloat16),
            preferred_element_type=jnp.float32).astype(jnp.bfloat16)

    out_ref[...] = lax.dot_general(
        hid_ref[...], w_ref[...].astype(jnp.bfloat16),
        (((1,), (1,)), ((), ())),
        preferred_element_type=jnp.float32,
    ) + b_ref[...]


def kernel(input_ids, embed_table, W_lm, b_lm):
    out = pl.pallas_call(
        _fused_body,
        grid=(NF,),
        in_specs=[
            pl.BlockSpec((B, 1), lambda j: (0, 0)),
            pl.BlockSpec((E, H), lambda j: (0, 0)),
            pl.BlockSpec((TVF, H), lambda j: (j, 0)),
            pl.BlockSpec((1, TVF), lambda j: (0, j)),
        ],
        out_specs=pl.BlockSpec((B, TVF), lambda j: (0, j)),
        out_shape=jax.ShapeDtypeStruct((B, V), jnp.float32),
        scratch_shapes=[pltpu.VMEM((B, H), jnp.bfloat16)],
    )(input_ids.reshape(B, 1).astype(jnp.int32), embed_table, W_lm,
      b_lm.reshape(1, V))
    return out.reshape(32, 32, V)
